# baseline (speedup 1.0000x reference)
import functools

import jax
import jax.numpy as jnp
from jax import lax
from jax.experimental import pallas as pl
from jax.experimental.pallas import tpu as pltpu
from jax.experimental.pallas import tpu_sc as plsc

B = 8
V = 2500
N = B * V
E = 60000
IMG_FEAT = 3840
ENC = 256
HID = 128
N_GCONV = 8

NTILES = 16
NHALF = N // 2
CH = 64
NPAD = 120832
SPR = 10008
RT = 632

BN_BOT = 400
BN = 1000


def _enc_body(e_ref, w0t_ref, w1t_ref, o0_ref, o1_ref):
    e = e_ref[...]
    z = jnp.zeros((HID - B, HID), jnp.float32)
    o0_ref[...] = jnp.concatenate(
        [jnp.dot(e, w0t_ref[...], preferred_element_type=jnp.float32), z], 0)
    o1_ref[...] = jnp.concatenate(
        [jnp.dot(e, w1t_ref[...], preferred_element_type=jnp.float32), z], 0)


def _enc_contrib(enc, w0t, w1t):
    return pl.pallas_call(
        _enc_body,
        out_shape=[
            jax.ShapeDtypeStruct((HID, HID), jnp.float32),
            jax.ShapeDtypeStruct((HID, HID), jnp.float32),
        ],
    )(enc, w0t, w1t)


def _bot0_body(x_ref, v_ref, w_ref, b_ref, w0h_ref, w1h_ref, w0v_ref,
               w1v_ref, e0_ref, e1_ref, b0_ref, b1_ref, o0_ref, o1_ref):
    j = pl.program_id(0)
    h = jnp.dot(x_ref[...], w_ref[...], preferred_element_type=jnp.float32)
    h = jnp.maximum(h + b_ref[...], 0.0)
    v = v_ref[...]
    mesh = (j * BN_BOT
            + jax.lax.broadcasted_iota(jnp.int32, (BN_BOT, HID), 0)) // V
    oneh = (mesh == jax.lax.broadcasted_iota(jnp.int32, (BN_BOT, HID), 1)
            ).astype(jnp.float32)
    o0_ref[...] = (jnp.dot(h, w0h_ref[...], preferred_element_type=jnp.float32)
                   + jnp.dot(v, w0v_ref[...], preferred_element_type=jnp.float32)
                   + jnp.dot(oneh, e0_ref[...], preferred_element_type=jnp.float32)
                   + b0_ref[...])
    o1_ref[...] = (jnp.dot(h, w1h_ref[...], preferred_element_type=jnp.float32)
                   + jnp.dot(v, w1v_ref[...], preferred_element_type=jnp.float32)
                   + jnp.dot(oneh, e1_ref[...], preferred_element_type=jnp.float32)
                   + b1_ref[...])


def _bot0(x, v, w, b, w0h, w1h, w0v, w1v, e0, e1, b0, b1):
    def full(shape):
        return pl.BlockSpec(shape, lambda j: (0, 0))
    return pl.pallas_call(
        _bot0_body,
        grid=(N // BN_BOT,),
        in_specs=[
            pl.BlockSpec((BN_BOT, IMG_FEAT), lambda j: (j, 0)),
            pl.BlockSpec((BN_BOT, 8), lambda j: (j, 0)),
            full((IMG_FEAT, HID)), full((1, HID)),
            full((HID, HID)), full((HID, HID)),
            full((8, HID)), full((8, HID)),
            full((HID, HID)), full((HID, HID)),
            full((1, HID)), full((1, HID)),
        ],
        out_specs=[
            pl.BlockSpec((BN_BOT, HID), lambda j: (j, 0)),
            pl.BlockSpec((BN_BOT, HID), lambda j: (j, 0)),
        ],
        out_shape=[
            jax.ShapeDtypeStruct((N, HID), jnp.float32),
            jax.ShapeDtypeStruct((N, HID), jnp.float32),
        ],
    )(x, v, w, b, w0h, w1h, w0v, w1v, e0, e1, b0, b1)


def _relu_mm_pair_body(w0x_ref, agg_ref, w0_ref, w1_ref, b0_ref, b1_ref,
                       o0_ref, o1_ref):
    x = jnp.maximum(w0x_ref[...] + agg_ref[...], 0.0)
    o0_ref[...] = jnp.dot(x, w0_ref[...],
                          preferred_element_type=jnp.float32) + b0_ref[...]
    o1_ref[...] = jnp.dot(x, w1_ref[...],
                          preferred_element_type=jnp.float32) + b1_ref[...]


def _relu_mm_pair(w0x, agg, w0, w1, b0, b1):
    return pl.pallas_call(
        _relu_mm_pair_body,
        grid=(N // BN,),
        in_specs=[
            pl.BlockSpec((BN, HID), lambda j: (j, 0)),
            pl.BlockSpec((BN, HID), lambda j: (j, 0)),
            pl.BlockSpec((HID, HID), lambda j: (0, 0)),
            pl.BlockSpec((HID, HID), lambda j: (0, 0)),
            pl.BlockSpec((1, HID), lambda j: (0, 0)),
            pl.BlockSpec((1, HID), lambda j: (0, 0)),
        ],
        out_specs=[
            pl.BlockSpec((BN, HID), lambda j: (j, 0)),
            pl.BlockSpec((BN, HID), lambda j: (j, 0)),
        ],
        out_shape=[
            jax.ShapeDtypeStruct((N, HID), jnp.float32),
            jax.ShapeDtypeStruct((N, HID), jnp.float32),
        ],
    )(w0x, agg, w0, w1, b0, b1)


def _final_body(w0x_ref, agg_ref, w_ref, b_ref, o_ref):
    x = jnp.maximum(w0x_ref[...] + agg_ref[...], 0.0)
    o_ref[...] = jnp.dot(x, w_ref[...],
                         preferred_element_type=jnp.float32) + b_ref[...]


def _final(w0x, agg, w, b):
    return pl.pallas_call(
        _final_body,
        grid=(N // BN,),
        in_specs=[
            pl.BlockSpec((BN, HID), lambda j: (j, 0)),
            pl.BlockSpec((BN, HID), lambda j: (j, 0)),
            pl.BlockSpec((HID, HID), lambda j: (0, 0)),
            pl.BlockSpec((1, HID), lambda j: (0, 0)),
        ],
        out_specs=pl.BlockSpec((BN, HID), lambda j: (j, 0)),
        out_shape=jax.ShapeDtypeStruct((N, HID), jnp.float32),
    )(w0x, agg, w, b)


NBUF = 6


@functools.cache
def _make_edge_agg_kernel():
    mesh = plsc.VectorSubcoreMesh(core_axis_name="c", subcore_axis_name="s",
                                  num_cores=2, num_subcores=NTILES)
    return functools.partial(
        pl.kernel,
        out_type=jax.ShapeDtypeStruct((N, HID), jnp.float32),
        mesh=mesh,
        scratch_types=(
            [pltpu.VMEM((CH,), jnp.int32)] * (2 * NBUF)
            + [pltpu.VMEM((CH, HID), jnp.float32)] * NBUF
            + [pltpu.VMEM((16,), jnp.int32)]
            + [pltpu.VMEM_SHARED((SPR, HID), jnp.float32)]
            + [pltpu.SemaphoreType.DMA] * (3 * NBUF)
        ),
    )(_edge_agg_body)


def _edge_agg_body(w1x_hbm, src_hbm, dst_hbm, meta_hbm, out_hbm, *scratch):
    src_v = scratch[0:NBUF]
    dst_v = scratch[NBUF:2 * NBUF]
    rows = scratch[2 * NBUF:3 * NBUF]
    meta_v = scratch[3 * NBUF]
    acc = scratch[3 * NBUF + 1]
    semi = scratch[3 * NBUF + 2:3 * NBUF + 2 + NBUF]
    semg = scratch[3 * NBUF + 2 + NBUF:3 * NBUF + 2 + 2 * NBUF]
    sems = scratch[3 * NBUF + 2 + 2 * NBUF:3 * NBUF + 2 + 3 * NBUF]
    c = lax.axis_index("c")
    s = lax.axis_index("s")

    pltpu.sync_copy(meta_hbm, meta_v)
    m0 = meta_v[...][0]
    nch = NPAD // CH
    k_lo = jnp.where(c == 0, 0, m0 // CH)
    k_hi = jnp.where(c == 0, (m0 + CH - 1) // CH, nch)
    cnt = jnp.maximum(0, (k_hi - k_lo - s + NTILES - 1) // NTILES)

    def idx_load(b, i):
        base = (k_lo + s + i * NTILES) * CH
        pltpu.async_copy(src_hbm.at[pl.ds(base, CH)], src_v[b], semi[b])
        pltpu.async_copy(dst_hbm.at[pl.ds(c * NPAD + base, CH)],
                         dst_v[b], semi[b])

    def idx_wait(b):
        pltpu.make_async_copy(src_hbm.at[pl.ds(0, CH)], src_v[b],
                              semi[b]).wait()
        pltpu.make_async_copy(src_hbm.at[pl.ds(0, CH)], dst_v[b],
                              semi[b]).wait()

    for b in range(NBUF):
        @pl.when(b < cnt)
        def _():
            idx_load(b, b)

    zero16 = jnp.zeros((16,), jnp.float32)

    def zfill(i, carry):
        for j in range(HID // 16):
            rows[0][i, pl.ds(j * 16, 16)] = zero16
        return carry

    lax.fori_loop(0, CH, zfill, 0)

    _NZC = -(-RT // CH)
    _LFULL = (SPR - (NTILES - 1) * RT) // CH
    _LREM = (SPR - (NTILES - 1) * RT) - _LFULL * CH

    @pl.when(s < NTILES - 1)
    def _():
        for r in range(_NZC):
            pltpu.sync_copy(rows[0], acc.at[pl.ds(s * RT + r * CH, CH)])

    @pl.when(s == NTILES - 1)
    def _():
        for r in range(_LFULL):
            pltpu.sync_copy(rows[0], acc.at[pl.ds(s * RT + r * CH, CH)])
        pltpu.sync_copy(rows[0].at[pl.ds(0, _LREM)],
                        acc.at[pl.ds(s * RT + _LFULL * CH, _LREM)])

    plsc.subcore_barrier()

    for b in range(NBUF):
        @pl.when(b < cnt)
        def _():
            idx_wait(b)
            pltpu.async_copy(w1x_hbm.at[src_v[b]], rows[b], semg[b])

    n_rounds = (cnt + NBUF - 1) // NBUF

    def round_fn(r, carry):
        k0 = r * NBUF
        for b in range(NBUF):
            @pl.when(k0 + b < cnt)
            def _():
                pltpu.make_async_copy(
                    w1x_hbm.at[src_v[b]], rows[b], semg[b]).wait()
                pltpu.async_copy(rows[b], acc.at[dst_v[b]], sems[b], add=True)

                @pl.when(k0 + b + NBUF < cnt)
                def _():
                    base = (k_lo + s + (k0 + b + NBUF) * NTILES) * CH
                    pltpu.async_copy(src_hbm.at[pl.ds(base, CH)],
                                     src_v[b], semi[b])
        for b in range(NBUF):
            @pl.when(k0 + b < cnt)
            def _():
                pltpu.make_async_copy(
                    rows[b], acc.at[dst_v[b]], sems[b]).wait()

                @pl.when(k0 + b + NBUF < cnt)
                def _():
                    base = (k_lo + s + (k0 + b + NBUF) * NTILES) * CH
                    pltpu.async_copy(dst_hbm.at[pl.ds(c * NPAD + base, CH)],
                                     dst_v[b], semi[b])
                    idx_wait(b)
                    pltpu.async_copy(w1x_hbm.at[src_v[b]], rows[b], semg[b])
        return carry

    lax.fori_loop(0, n_rounds, round_fn, 0)
    plsc.subcore_barrier()

    @pl.when(s < NTILES - 1)
    def _():
        pltpu.sync_copy(acc.at[pl.ds(s * RT, RT)],
                        out_hbm.at[pl.ds(c * NHALF + s * RT, RT)])

    @pl.when(s == NTILES - 1)
    def _():
        last = NHALF - (NTILES - 1) * RT
        pltpu.sync_copy(acc.at[pl.ds(s * RT, last)],
                        out_hbm.at[pl.ds(c * NHALF + s * RT, last)])


def _edge_agg(w1x, src1, dst2, meta):
    return _make_edge_agg_kernel()(w1x, src1, dst2, meta)


def kernel(vert_align_feats, verts_packed, image_encodings, params, edges):
    p = params

    e = edges.astype(jnp.int32)
    src = jnp.concatenate([e[:, 1], e[:, 0]])
    dst = jnp.concatenate([e[:, 0], e[:, 1]])
    src = jnp.pad(src, (0, NPAD - 2 * E))
    dst = jnp.pad(dst, (0, NPAD - 2 * E), constant_values=N)
    half = (dst >= NHALF).astype(jnp.int32)
    perm = jnp.argsort(half)
    src1 = src[perm]
    dsts = dst[perm]
    m0 = NPAD - jnp.sum(half)
    dst0 = jnp.where(dsts < NHALF, dsts, NHALF)
    dst1 = jnp.where(dsts >= NHALF, dsts - NHALF, NHALF)
    dst2 = jnp.concatenate([dst0, dst1])
    meta = jnp.full((16,), m0, jnp.int32)

    e0, e1 = _enc_contrib(image_encodings,
                          p['W0_0'][HID + 3:], p['W1_0'][HID + 3:])
    vafb = vert_align_feats.astype(jnp.bfloat16)
    wbb = p['Wb'].astype(jnp.bfloat16)
    v8 = jnp.pad(verts_packed, ((0, 0), (0, 5)))
    w0v = jnp.pad(p['W0_0'][HID:HID + 3], ((0, 5), (0, 0)))
    w1v = jnp.pad(p['W1_0'][HID:HID + 3], ((0, 5), (0, 0)))
    w0x, w1x = _bot0(vafb, v8, wbb, p['bb'].reshape(1, HID),
                     p['W0_0'][:HID], p['W1_0'][:HID], w0v, w1v, e0, e1,
                     p['b0_0'].reshape(1, HID), p['b1_0'].reshape(1, HID))

    for i in range(1, N_GCONV):
        agg = _edge_agg(w1x, src1, dst2, meta)
        w0x, w1x = _relu_mm_pair(w0x, agg,
                                 p[f'W0_{i}'], p[f'W1_{i}'],
                                 p[f'b0_{i}'].reshape(1, HID),
                                 p[f'b1_{i}'].reshape(1, HID))

    agg = _edge_agg(w1x, src1, dst2, meta)
    wo = jnp.pad(p['Wo'], ((0, 0), (0, HID - 3)))
    bo = jnp.pad(p['bo'], (0, HID - 3)).reshape(1, HID)
    out = _final(w0x, agg, wo, bo)
    return out[:, :3]


# speedup vs baseline: 1.1078x; 1.1078x over previous
import functools

import jax
import jax.numpy as jnp
from jax import lax
from jax.experimental import pallas as pl
from jax.experimental.pallas import tpu as pltpu
from jax.experimental.pallas import tpu_sc as plsc

B = 8
V = 2500
N = B * V
E = 60000
IMG_FEAT = 3840
ENC = 256
HID = 128
N_GCONV = 8

NTILES = 16
NHALF = N // 2
CH = 64
NPAD = 120832
SPR = 10008
RT = 632

BN_BOT = 400
BN = 1000


def _enc_body(e_ref, w0t_ref, w1t_ref, o0_ref, o1_ref):
    e = e_ref[...]
    z = jnp.zeros((HID - B, HID), jnp.float32)
    o0_ref[...] = jnp.concatenate(
        [jnp.dot(e, w0t_ref[...], preferred_element_type=jnp.float32), z], 0)
    o1_ref[...] = jnp.concatenate(
        [jnp.dot(e, w1t_ref[...], preferred_element_type=jnp.float32), z], 0)


def _enc_contrib(enc, w0t, w1t):
    return pl.pallas_call(
        _enc_body,
        out_shape=[
            jax.ShapeDtypeStruct((HID, HID), jnp.float32),
            jax.ShapeDtypeStruct((HID, HID), jnp.float32),
        ],
    )(enc, w0t, w1t)


def _bot0_body(x_ref, v_ref, w_ref, b_ref, w0h_ref, w1h_ref, w0v_ref,
               w1v_ref, e0_ref, e1_ref, b0_ref, b1_ref, o0_ref, o1_ref):
    j = pl.program_id(0)
    h = jnp.dot(x_ref[...], w_ref[...], preferred_element_type=jnp.float32)
    h = jnp.maximum(h + b_ref[...], 0.0)
    v = v_ref[...]
    mesh = (j * BN_BOT
            + jax.lax.broadcasted_iota(jnp.int32, (BN_BOT, HID), 0)) // V
    oneh = (mesh == jax.lax.broadcasted_iota(jnp.int32, (BN_BOT, HID), 1)
            ).astype(jnp.float32)
    o0_ref[...] = (jnp.dot(h, w0h_ref[...], preferred_element_type=jnp.float32)
                   + jnp.dot(v, w0v_ref[...], preferred_element_type=jnp.float32)
                   + jnp.dot(oneh, e0_ref[...], preferred_element_type=jnp.float32)
                   + b0_ref[...])
    o1_ref[...] = (jnp.dot(h, w1h_ref[...], preferred_element_type=jnp.float32)
                   + jnp.dot(v, w1v_ref[...], preferred_element_type=jnp.float32)
                   + jnp.dot(oneh, e1_ref[...], preferred_element_type=jnp.float32)
                   + b1_ref[...])


def _bot0(x, v, w, b, w0h, w1h, w0v, w1v, e0, e1, b0, b1):
    def full(shape):
        return pl.BlockSpec(shape, lambda j: (0, 0))
    return pl.pallas_call(
        _bot0_body,
        grid=(N // BN_BOT,),
        in_specs=[
            pl.BlockSpec((BN_BOT, IMG_FEAT), lambda j: (j, 0)),
            pl.BlockSpec((BN_BOT, 8), lambda j: (j, 0)),
            full((IMG_FEAT, HID)), full((1, HID)),
            full((HID, HID)), full((HID, HID)),
            full((8, HID)), full((8, HID)),
            full((HID, HID)), full((HID, HID)),
            full((1, HID)), full((1, HID)),
        ],
        out_specs=[
            pl.BlockSpec((BN_BOT, HID), lambda j: (j, 0)),
            pl.BlockSpec((BN_BOT, HID), lambda j: (j, 0)),
        ],
        out_shape=[
            jax.ShapeDtypeStruct((N, HID), jnp.float32),
            jax.ShapeDtypeStruct((N, HID), jnp.float32),
        ],
    )(x, v, w, b, w0h, w1h, w0v, w1v, e0, e1, b0, b1)


def _relu_mm_pair_body(w0x_ref, agg_ref, w0_ref, w1_ref, b0_ref, b1_ref,
                       o0_ref, o1_ref):
    x = jnp.maximum(w0x_ref[...] + agg_ref[...], 0.0)
    o0_ref[...] = jnp.dot(x, w0_ref[...],
                          preferred_element_type=jnp.float32) + b0_ref[...]
    o1_ref[...] = jnp.dot(x, w1_ref[...],
                          preferred_element_type=jnp.float32) + b1_ref[...]


def _relu_mm_pair(w0x, agg, w0, w1, b0, b1):
    return pl.pallas_call(
        _relu_mm_pair_body,
        grid=(N // BN,),
        in_specs=[
            pl.BlockSpec((BN, HID), lambda j: (j, 0)),
            pl.BlockSpec((BN, HID), lambda j: (j, 0)),
            pl.BlockSpec((HID, HID), lambda j: (0, 0)),
            pl.BlockSpec((HID, HID), lambda j: (0, 0)),
            pl.BlockSpec((1, HID), lambda j: (0, 0)),
            pl.BlockSpec((1, HID), lambda j: (0, 0)),
        ],
        out_specs=[
            pl.BlockSpec((BN, HID), lambda j: (j, 0)),
            pl.BlockSpec((BN, HID), lambda j: (j, 0)),
        ],
        out_shape=[
            jax.ShapeDtypeStruct((N, HID), jnp.float32),
            jax.ShapeDtypeStruct((N, HID), jnp.float32),
        ],
    )(w0x, agg, w0, w1, b0, b1)


def _final_body(w0x_ref, agg_ref, w_ref, b_ref, o_ref):
    x = jnp.maximum(w0x_ref[...] + agg_ref[...], 0.0)
    o_ref[...] = jnp.dot(x, w_ref[...],
                         preferred_element_type=jnp.float32) + b_ref[...]


def _final(w0x, agg, w, b):
    return pl.pallas_call(
        _final_body,
        grid=(N // BN,),
        in_specs=[
            pl.BlockSpec((BN, HID), lambda j: (j, 0)),
            pl.BlockSpec((BN, HID), lambda j: (j, 0)),
            pl.BlockSpec((HID, HID), lambda j: (0, 0)),
            pl.BlockSpec((1, HID), lambda j: (0, 0)),
        ],
        out_specs=pl.BlockSpec((BN, HID), lambda j: (j, 0)),
        out_shape=jax.ShapeDtypeStruct((N, HID), jnp.float32),
    )(w0x, agg, w, b)


NBUF = 6


@functools.cache
def _make_edge_agg_kernel():
    mesh = plsc.VectorSubcoreMesh(core_axis_name="c", subcore_axis_name="s",
                                  num_cores=2, num_subcores=NTILES)
    return functools.partial(
        pl.kernel,
        out_type=jax.ShapeDtypeStruct((N, HID), jnp.float32),
        mesh=mesh,
        scratch_types=(
            [pltpu.VMEM((CH,), jnp.int32)] * (2 * NBUF)
            + [pltpu.VMEM((CH, HID), jnp.float32)] * NBUF
            + [pltpu.VMEM((16,), jnp.int32)]
            + [pltpu.VMEM_SHARED((SPR, HID), jnp.float32)]
            + [pltpu.SemaphoreType.DMA] * (3 * NBUF)
        ),
    )(_edge_agg_body)


def _edge_agg_body(w1x_hbm, src_hbm, dst_hbm, meta_hbm, out_hbm, *scratch):
    src_v = scratch[0:NBUF]
    dst_v = scratch[NBUF:2 * NBUF]
    rows = scratch[2 * NBUF:3 * NBUF]
    meta_v = scratch[3 * NBUF]
    acc = scratch[3 * NBUF + 1]
    semi = scratch[3 * NBUF + 2:3 * NBUF + 2 + NBUF]
    semg = scratch[3 * NBUF + 2 + NBUF:3 * NBUF + 2 + 2 * NBUF]
    sems = scratch[3 * NBUF + 2 + 2 * NBUF:3 * NBUF + 2 + 3 * NBUF]
    c = lax.axis_index("c")
    s = lax.axis_index("s")

    pltpu.sync_copy(meta_hbm, meta_v)
    m0 = meta_v[...][0]
    nch = NPAD // CH
    k_lo = jnp.where(c == 0, 0, m0 // CH)
    k_hi = jnp.where(c == 0, (m0 + CH - 1) // CH, nch)
    cnt = jnp.maximum(0, (k_hi - k_lo - s + NTILES - 1) // NTILES)

    def idx_load(b, i):
        base = (k_lo + s + i * NTILES) * CH
        pltpu.async_copy(src_hbm.at[pl.ds(base, CH)], src_v[b], semi[b])
        pltpu.async_copy(dst_hbm.at[pl.ds(c * NPAD + base, CH)],
                         dst_v[b], semi[b])

    def idx_wait(b):
        pltpu.make_async_copy(src_hbm.at[pl.ds(0, CH)], src_v[b],
                              semi[b]).wait()
        pltpu.make_async_copy(src_hbm.at[pl.ds(0, CH)], dst_v[b],
                              semi[b]).wait()

    for b in range(NBUF):
        @pl.when(b < cnt)
        def _():
            idx_load(b, b)

    zero16 = jnp.zeros((16,), jnp.float32)

    def zfill(i, carry):
        for j in range(HID // 16):
            rows[0][i, pl.ds(j * 16, 16)] = zero16
        return carry

    lax.fori_loop(0, CH, zfill, 0)

    _NZC = -(-RT // CH)
    _LFULL = (SPR - (NTILES - 1) * RT) // CH
    _LREM = (SPR - (NTILES - 1) * RT) - _LFULL * CH

    @pl.when(s < NTILES - 1)
    def _():
        for r in range(_NZC):
            pltpu.sync_copy(rows[0], acc.at[pl.ds(s * RT + r * CH, CH)])

    @pl.when(s == NTILES - 1)
    def _():
        for r in range(_LFULL):
            pltpu.sync_copy(rows[0], acc.at[pl.ds(s * RT + r * CH, CH)])
        pltpu.sync_copy(rows[0].at[pl.ds(0, _LREM)],
                        acc.at[pl.ds(s * RT + _LFULL * CH, _LREM)])

    plsc.subcore_barrier()

    for b in range(NBUF):
        @pl.when(b < cnt)
        def _():
            idx_wait(b)
            pltpu.async_copy(w1x_hbm.at[src_v[b]], rows[b], semg[b])

    n_rounds = (cnt + NBUF - 1) // NBUF

    def round_fn(r, carry):
        k0 = r * NBUF
        for b in range(NBUF):
            @pl.when(k0 + b < cnt)
            def _():
                pltpu.make_async_copy(
                    w1x_hbm.at[src_v[b]], rows[b], semg[b]).wait()
                pltpu.async_copy(rows[b], acc.at[dst_v[b]], sems[b], add=True)

                @pl.when(k0 + b + NBUF < cnt)
                def _():
                    base = (k_lo + s + (k0 + b + NBUF) * NTILES) * CH
                    pltpu.async_copy(src_hbm.at[pl.ds(base, CH)],
                                     src_v[b], semi[b])
        for b in range(NBUF):
            @pl.when(k0 + b < cnt)
            def _():
                pltpu.make_async_copy(
                    rows[b], acc.at[dst_v[b]], sems[b]).wait()

                @pl.when(k0 + b + NBUF < cnt)
                def _():
                    base = (k_lo + s + (k0 + b + NBUF) * NTILES) * CH
                    pltpu.async_copy(dst_hbm.at[pl.ds(c * NPAD + base, CH)],
                                     dst_v[b], semi[b])
                    idx_wait(b)
                    pltpu.async_copy(w1x_hbm.at[src_v[b]], rows[b], semg[b])
        return carry

    lax.fori_loop(0, n_rounds, round_fn, 0)
    plsc.subcore_barrier()

    @pl.when(s < NTILES - 1)
    def _():
        pltpu.sync_copy(acc.at[pl.ds(s * RT, RT)],
                        out_hbm.at[pl.ds(c * NHALF + s * RT, RT)])

    @pl.when(s == NTILES - 1)
    def _():
        last = NHALF - (NTILES - 1) * RT
        pltpu.sync_copy(acc.at[pl.ds(s * RT, last)],
                        out_hbm.at[pl.ds(c * NHALF + s * RT, last)])


def _edge_agg(w1x, src1, dst2, meta):
    return _make_edge_agg_kernel()(w1x, src1, dst2, meta)


def kernel(vert_align_feats, verts_packed, image_encodings, params, edges):
    p = params

    e = edges.astype(jnp.int32)
    src = jnp.concatenate([e[:, 1], e[:, 0]])
    dst = jnp.concatenate([e[:, 0], e[:, 1]])
    src = jnp.pad(src, (0, NPAD - 2 * E))
    dst = jnp.pad(dst, (0, NPAD - 2 * E), constant_values=N)
    half = (dst >= NHALF).astype(jnp.int32)
    perm = jnp.argsort(half)
    src1 = src[perm]
    dsts = dst[perm]
    m0 = NPAD - jnp.sum(half)
    dst0 = jnp.where(dsts < NHALF, dsts, NHALF)
    dst1 = jnp.where(dsts >= NHALF, dsts - NHALF, NHALF)
    dst2 = jnp.concatenate([dst0, dst1])
    meta = jnp.full((16,), m0, jnp.int32)

    e0, e1 = _enc_contrib(image_encodings,
                          p['W0_0'][HID + 3:], p['W1_0'][HID + 3:])
    v8 = jnp.pad(verts_packed, ((0, 0), (0, 5)))
    w0v = jnp.pad(p['W0_0'][HID:HID + 3], ((0, 5), (0, 0)))
    w1v = jnp.pad(p['W1_0'][HID:HID + 3], ((0, 5), (0, 0)))
    w0x, w1x = _bot0(vert_align_feats, v8, p['Wb'], p['bb'].reshape(1, HID),
                     p['W0_0'][:HID], p['W1_0'][:HID], w0v, w1v, e0, e1,
                     p['b0_0'].reshape(1, HID), p['b1_0'].reshape(1, HID))

    for i in range(1, N_GCONV):
        agg = _edge_agg(w1x, src1, dst2, meta)
        w0x, w1x = _relu_mm_pair(w0x, agg,
                                 p[f'W0_{i}'], p[f'W1_{i}'],
                                 p[f'b0_{i}'].reshape(1, HID),
                                 p[f'b1_{i}'].reshape(1, HID))

    agg = _edge_agg(w1x, src1, dst2, meta)
    wo = jnp.pad(p['Wo'], ((0, 0), (0, HID - 3)))
    bo = jnp.pad(p['bo'], (0, HID - 3)).reshape(1, HID)
    out = _final(w0x, agg, wo, bo)
    return out[:, :3]
